# Initial kernel scaffold; baseline (speedup 1.0000x reference)
import jax
import jax.numpy as jnp
from jax.experimental import pallas as pl


def kernel(rad1_w, theta1_w, rad2_w, theta2_w, x_input, y_target, y_noise):
    raise NotImplementedError("write your pallas kernel here")


# speedup vs baseline: 1.8584x; 1.8584x over previous
import functools
import math

import jax
import jax.numpy as jnp
from jax import lax
from jax.experimental import pallas as pl
from jax.experimental.pallas import tpu as pltpu
from jax.experimental.pallas import tpu_sc as plsc


def _make_sc_gather(nx, ny, nn_pad):
    info = plsc.get_sparse_core_info()
    nc, ns = info.num_cores, info.num_subcores
    nw = nc * ns
    assert nx % (8 * nw) == 0 and ny % (8 * nw) == 0 and nn_pad % (8 * nw) == 0
    xc, yc, nnc = nx // nw, ny // nw, nn_pad // nw

    mesh = plsc.VectorSubcoreMesh(core_axis_name="c", subcore_axis_name="s")

    @functools.partial(
        pl.kernel,
        mesh=mesh,
        out_type=[
            jax.ShapeDtypeStruct((nx,), jnp.float32),
            jax.ShapeDtypeStruct((nx,), jnp.float32),
            jax.ShapeDtypeStruct((ny,), jnp.float32),
            jax.ShapeDtypeStruct((ny,), jnp.float32),
            jax.ShapeDtypeStruct((nn_pad,), jnp.float32),
            jax.ShapeDtypeStruct((nn_pad,), jnp.float32),
        ],
        scratch_types=[
            pltpu.VMEM((xc,), jnp.int32),
            pltpu.VMEM((xc,), jnp.float32),
            pltpu.VMEM((nnc,), jnp.int32),
            pltpu.VMEM((nnc,), jnp.float32),
            pltpu.SemaphoreType.DMA,
        ],
    )
    def gather(rad1, theta1, rad2, theta2, x, y, yn,
               o_r1x, o_t1x, o_r2y, o_t2y, o_r2n, o_t2n,
               idx_v, buf_v, idxn_v, bufn_v, sem):
        wid = lax.axis_index("s") * nc + lax.axis_index("c")

        bx = wid * xc
        pltpu.sync_copy(x.at[pl.ds(bx, xc)], idx_v)
        pltpu.async_copy(rad1.at[idx_v], buf_v, sem).wait()
        pltpu.sync_copy(buf_v, o_r1x.at[pl.ds(bx, xc)])
        pltpu.async_copy(theta1.at[idx_v], buf_v, sem).wait()
        pltpu.sync_copy(buf_v, o_t1x.at[pl.ds(bx, xc)])

        by = wid * yc
        pltpu.sync_copy(y.at[pl.ds(by, yc)], idx_v)
        pltpu.async_copy(rad2.at[idx_v], buf_v, sem).wait()
        pltpu.sync_copy(buf_v, o_r2y.at[pl.ds(by, yc)])
        pltpu.async_copy(theta2.at[idx_v], buf_v, sem).wait()
        pltpu.sync_copy(buf_v, o_t2y.at[pl.ds(by, yc)])

        bn = wid * nnc
        pltpu.sync_copy(yn.at[pl.ds(bn, nnc)], idxn_v)
        pltpu.async_copy(rad2.at[idxn_v], bufn_v, sem).wait()
        pltpu.sync_copy(bufn_v, o_r2n.at[pl.ds(bn, nnc)])
        pltpu.async_copy(theta2.at[idxn_v], bufn_v, sem).wait()
        pltpu.sync_copy(bufn_v, o_t2n.at[pl.ds(bn, nnc)])

    return gather


def _atanh(x):
    return 0.5 * jnp.log(jnp.abs((1.0 + x) / (1.0 - x)))


def _tc_body(nn, ax_c, tx_c, ax_r, tx_r, by_r, ty_r, bn_c, tn_c, out_ref, s_ref):
    @pl.when(pl.program_id(0) == 0)
    def _():
        a_r = 4.0 * _atanh(ax_r[...])
        u0r = a_r * jnp.cos(tx_r[...])
        u1r = a_r * jnp.sin(tx_r[...])
        b_n = _atanh(bn_c[...])
        vn0 = b_n * jnp.cos(tn_c[...])
        vn1 = b_n * jnp.sin(tn_c[...])
        zn = vn0 * u0r + vn1 * u1r
        sg = 1.0 / (1.0 + jnp.exp(-zn))
        row = lax.broadcasted_iota(jnp.int32, zn.shape, 0)
        s_ref[0, 0] = jnp.sum(jnp.where(row < nn, sg, 0.0))

    a = 4.0 * _atanh(ax_c[...])
    u0 = a * jnp.cos(tx_c[...])
    u1 = a * jnp.sin(tx_c[...])
    b = _atanh(by_r[...])
    v0 = b * jnp.cos(ty_r[...])
    v1 = b * jnp.sin(ty_r[...])
    z = u0 * v0 + u1 * v1
    out_ref[...] = (-s_ref[0, 0]) - 1.0 / (1.0 + jnp.exp(-z))


def kernel(rad1_w, theta1_w, rad2_w, theta2_w, x_input, y_target, y_noise):
    nx = x_input.shape[0]
    ny = y_target.shape[0]
    nn = y_noise.shape[0]
    nn_pad = max(256, -(-nn // 256) * 256)

    x = x_input.astype(jnp.int32)
    y = y_target.astype(jnp.int32)
    yn = jnp.zeros((nn_pad,), jnp.int32).at[:nn].set(y_noise.astype(jnp.int32))

    r1 = rad1_w.reshape(-1)
    t1 = theta1_w.reshape(-1)
    r2 = rad2_w.reshape(-1)
    t2 = theta2_w.reshape(-1)

    g_r1x, g_t1x, g_r2y, g_t2y, g_r2n, g_t2n = _make_sc_gather(nx, ny, nn_pad)(
        r1, t1, r2, t2, x, y, yn)

    TR = 512
    assert nx % TR == 0
    grid = (nx // TR,)

    col = lambda n: pl.BlockSpec((n, 1), lambda i: (0, 0))
    out = pl.pallas_call(
        functools.partial(_tc_body, nn),
        grid=grid,
        in_specs=[
            pl.BlockSpec((TR, 1), lambda i: (i, 0)),
            pl.BlockSpec((TR, 1), lambda i: (i, 0)),
            pl.BlockSpec((1, nx), lambda i: (0, 0)),
            pl.BlockSpec((1, nx), lambda i: (0, 0)),
            pl.BlockSpec((1, ny), lambda i: (0, 0)),
            pl.BlockSpec((1, ny), lambda i: (0, 0)),
            col(nn_pad),
            col(nn_pad),
        ],
        out_specs=pl.BlockSpec((TR, ny), lambda i: (i, 0)),
        out_shape=jax.ShapeDtypeStruct((nx, ny), jnp.float32),
        scratch_shapes=[pltpu.SMEM((1, 1), jnp.float32)],
    )(
        g_r1x.reshape(nx, 1), g_t1x.reshape(nx, 1),
        g_r1x.reshape(1, nx), g_t1x.reshape(1, nx),
        g_r2y.reshape(1, ny), g_t2y.reshape(1, ny),
        g_r2n.reshape(nn_pad, 1), g_t2n.reshape(nn_pad, 1),
    )
    return out


# speedup vs baseline: 1.9338x; 1.0406x over previous
import functools
import math

import jax
import jax.numpy as jnp
from jax import lax
from jax.experimental import pallas as pl
from jax.experimental.pallas import tpu as pltpu
from jax.experimental.pallas import tpu_sc as plsc


def _make_sc_gather(nx, ny, nn_pad):
    info = plsc.get_sparse_core_info()
    nc, ns = info.num_cores, info.num_subcores
    nw = nc * ns
    assert nx % (8 * nw) == 0 and ny % (8 * nw) == 0 and nn_pad % (8 * nw) == 0
    xc, yc, nnc = nx // nw, ny // nw, nn_pad // nw

    mesh = plsc.VectorSubcoreMesh(core_axis_name="c", subcore_axis_name="s")

    @functools.partial(
        pl.kernel,
        mesh=mesh,
        out_type=[
            jax.ShapeDtypeStruct((nx,), jnp.float32),
            jax.ShapeDtypeStruct((nx,), jnp.float32),
            jax.ShapeDtypeStruct((ny,), jnp.float32),
            jax.ShapeDtypeStruct((ny,), jnp.float32),
            jax.ShapeDtypeStruct((nn_pad,), jnp.float32),
            jax.ShapeDtypeStruct((nn_pad,), jnp.float32),
        ],
        scratch_types=[
            pltpu.VMEM((xc,), jnp.int32),
            pltpu.VMEM((xc,), jnp.float32),
            pltpu.VMEM((nnc,), jnp.int32),
            pltpu.VMEM((nnc,), jnp.float32),
            pltpu.SemaphoreType.DMA,
        ],
    )
    def gather(rad1, theta1, rad2, theta2, x, y, yn,
               o_r1x, o_t1x, o_r2y, o_t2y, o_r2n, o_t2n,
               idx_v, buf_v, idxn_v, bufn_v, sem):
        wid = lax.axis_index("s") * nc + lax.axis_index("c")

        bx = wid * xc
        pltpu.sync_copy(x.at[pl.ds(bx, xc)], idx_v)
        pltpu.async_copy(rad1.at[idx_v], buf_v, sem).wait()
        pltpu.sync_copy(buf_v, o_r1x.at[pl.ds(bx, xc)])
        pltpu.async_copy(theta1.at[idx_v], buf_v, sem).wait()
        pltpu.sync_copy(buf_v, o_t1x.at[pl.ds(bx, xc)])

        by = wid * yc
        pltpu.sync_copy(y.at[pl.ds(by, yc)], idx_v)
        pltpu.async_copy(rad2.at[idx_v], buf_v, sem).wait()
        pltpu.sync_copy(buf_v, o_r2y.at[pl.ds(by, yc)])
        pltpu.async_copy(theta2.at[idx_v], buf_v, sem).wait()
        pltpu.sync_copy(buf_v, o_t2y.at[pl.ds(by, yc)])

        bn = wid * nnc
        pltpu.sync_copy(yn.at[pl.ds(bn, nnc)], idxn_v)
        pltpu.async_copy(rad2.at[idxn_v], bufn_v, sem).wait()
        pltpu.sync_copy(bufn_v, o_r2n.at[pl.ds(bn, nnc)])
        pltpu.async_copy(theta2.at[idxn_v], bufn_v, sem).wait()
        pltpu.sync_copy(bufn_v, o_t2n.at[pl.ds(bn, nnc)])

    return gather


def _atanh(x):
    return 0.5 * jnp.log(jnp.abs((1.0 + x) / (1.0 - x)))


def _tc_body(nn, nx, ax_c, tx_c, ax_r, tx_r, by_r, ty_r, bn_c, tn_c, out_ref, s_ref):
    @pl.when(pl.program_id(0) == 0)
    def _():
        a_r = 2.0 * _atanh(ax_r[...])
        u0r = a_r * jnp.cos(tx_r[...])
        u1r = a_r * jnp.sin(tx_r[...])
        b_n = _atanh(bn_c[...])
        vn0 = b_n * jnp.cos(tn_c[...])
        vn1 = b_n * jnp.sin(tn_c[...])
        zn = vn0 * u0r + vn1 * u1r
        row = lax.broadcasted_iota(jnp.int32, zn.shape, 0)
        th = jnp.sum(jnp.where(row < nn, jnp.tanh(zn), 0.0))
        s_val = 0.5 * (nn * nx) + 0.5 * th
        s_ref[0, 0] = -s_val - 0.5

    a = 2.0 * _atanh(ax_c[...])
    u0 = a * jnp.cos(tx_c[...])
    u1 = a * jnp.sin(tx_c[...])
    b = _atanh(by_r[...])
    v0 = b * jnp.cos(ty_r[...])
    v1 = b * jnp.sin(ty_r[...])
    z = u0 * v0 + u1 * v1
    out_ref[...] = s_ref[0, 0] - 0.5 * jnp.tanh(z)


def kernel(rad1_w, theta1_w, rad2_w, theta2_w, x_input, y_target, y_noise):
    nx = x_input.shape[0]
    ny = y_target.shape[0]
    nn = y_noise.shape[0]
    nn_pad = max(256, -(-nn // 256) * 256)

    x = x_input.astype(jnp.int32)
    y = y_target.astype(jnp.int32)
    yn = jnp.zeros((nn_pad,), jnp.int32).at[:nn].set(y_noise.astype(jnp.int32))

    r1 = rad1_w.reshape(-1)
    t1 = theta1_w.reshape(-1)
    r2 = rad2_w.reshape(-1)
    t2 = theta2_w.reshape(-1)

    g_r1x, g_t1x, g_r2y, g_t2y, g_r2n, g_t2n = _make_sc_gather(nx, ny, nn_pad)(
        r1, t1, r2, t2, x, y, yn)

    TR = 512
    assert nx % TR == 0
    grid = (nx // TR,)

    col = lambda n: pl.BlockSpec((n, 1), lambda i: (0, 0))
    out = pl.pallas_call(
        functools.partial(_tc_body, nn, nx),
        grid=grid,
        in_specs=[
            pl.BlockSpec((TR, 1), lambda i: (i, 0)),
            pl.BlockSpec((TR, 1), lambda i: (i, 0)),
            pl.BlockSpec((1, nx), lambda i: (0, 0)),
            pl.BlockSpec((1, nx), lambda i: (0, 0)),
            pl.BlockSpec((1, ny), lambda i: (0, 0)),
            pl.BlockSpec((1, ny), lambda i: (0, 0)),
            col(nn_pad),
            col(nn_pad),
        ],
        out_specs=pl.BlockSpec((TR, ny), lambda i: (i, 0)),
        out_shape=jax.ShapeDtypeStruct((nx, ny), jnp.float32),
        scratch_shapes=[pltpu.SMEM((1, 1), jnp.float32)],
    )(
        g_r1x.reshape(nx, 1), g_t1x.reshape(nx, 1),
        g_r1x.reshape(1, nx), g_t1x.reshape(1, nx),
        g_r2y.reshape(1, ny), g_t2y.reshape(1, ny),
        g_r2n.reshape(nn_pad, 1), g_t2n.reshape(nn_pad, 1),
    )
    return out
